# Initial kernel scaffold; baseline (speedup 1.0000x reference)
import jax
import jax.numpy as jnp
from jax.experimental import pallas as pl


def kernel(x, Gw_sea, Ew_sea, Eb_sea, Gw_trend, Ew_trend, Eb_trend):
    raise NotImplementedError("write your pallas kernel here")


# speedup vs baseline: 1.0479x; 1.0479x over previous
import functools

import jax
import jax.numpy as jnp
from jax.experimental import pallas as pl

KERNEL = 25
TOPK = 2


def _decomp_gate_kernel(x_ref, gs_ref, gt_ref, st_ref, c_ref, pm_ref, *, V, L, E, B):
    b = pl.program_id(0)
    xb = x_ref[...]
    pad = (KERNEL - 1) // 2
    front = jnp.broadcast_to(xb[:, :1], (V, pad))
    back = jnp.broadcast_to(xb[:, -1:], (V, pad))
    xp = jnp.concatenate([front, xb, back], axis=1)
    mov = xp[:, 0:L]
    for d in range(1, KERNEL):
        mov = mov + xp[:, d:d + L]
    mov = mov * (1.0 / KERNEL)
    sea = xb - mov
    st_ref[0] = sea
    st_ref[1] = mov

    for m, (a, g_ref) in enumerate(((sea, gs_ref), (mov, gt_ref))):
        logits = jnp.dot(a, g_ref[...], preferred_element_type=jnp.float32)
        z = logits - jnp.max(logits, axis=-1, keepdims=True)
        ez = jnp.exp(z)
        probs = ez / jnp.sum(ez, axis=-1, keepdims=True)
        iota = jax.lax.broadcasted_iota(jnp.int32, (V, E), 1)
        m1 = jnp.max(probs, axis=-1, keepdims=True)
        sel1 = jnp.min(jnp.where(probs == m1, iota, E), axis=-1, keepdims=True)
        oh1 = iota == sel1
        masked = jnp.where(oh1, -1.0, probs)
        m2 = jnp.max(masked, axis=-1, keepdims=True)
        sel2 = jnp.min(jnp.where(masked == m2, iota, E), axis=-1, keepdims=True)
        oh2 = iota == sel2
        c_ref[m] = jnp.where(oh1 | oh2, probs, 0.0)
        if m == 1:
            scaled = probs * (1.0 / B)

            @pl.when(b == 0)
            def _():
                pm_ref[...] = scaled

            @pl.when(b > 0)
            def _():
                pm_ref[...] += scaled


def _expert_kernel(a_ref, w_ref, c_ref, b_ref, o_ref, *, m):
    e = pl.program_id(0)
    y = jax.lax.dot_general(
        a_ref[0], w_ref[0], (((1,), (1,)), ((), ())),
        preferred_element_type=jnp.float32)
    c_full = c_ref[m]
    lane = jax.lax.broadcasted_iota(jnp.int32, c_full.shape, 1)
    ccol = jnp.sum(jnp.where(lane == e, c_full, 0.0), axis=1, keepdims=True)
    b_full = b_ref[...]
    sub = jax.lax.broadcasted_iota(jnp.int32, b_full.shape, 0)
    brow = jnp.sum(jnp.where(sub == e, b_full, 0.0), axis=0, keepdims=True)
    contrib = ccol * (y + brow)

    @pl.when(e == 0)
    def _():
        o_ref[...] = contrib

    @pl.when(e > 0)
    def _():
        o_ref[...] += contrib


def kernel(x, Gw_sea, Ew_sea, Eb_sea, Gw_trend, Ew_trend, Eb_trend):
    B, L, V = x.shape
    E, P, _ = Ew_sea.shape
    T = B * V
    xt = jnp.transpose(x, (0, 2, 1)).reshape(T, L)

    st, c_all, pm = pl.pallas_call(
        functools.partial(_decomp_gate_kernel, V=V, L=L, E=E, B=B),
        grid=(B,),
        in_specs=[
            pl.BlockSpec((V, L), lambda b: (b, 0)),
            pl.BlockSpec((L, E), lambda b: (0, 0)),
            pl.BlockSpec((L, E), lambda b: (0, 0)),
        ],
        out_specs=[
            pl.BlockSpec((2, V, L), lambda b: (0, b, 0)),
            pl.BlockSpec((2, V, E), lambda b: (0, b, 0)),
            pl.BlockSpec((V, E), lambda b: (0, 0)),
        ],
        out_shape=[
            jax.ShapeDtypeStruct((2, T, L), jnp.float32),
            jax.ShapeDtypeStruct((2, T, E), jnp.float32),
            jax.ShapeDtypeStruct((V, E), jnp.float32),
        ],
    )(xt, Gw_sea.T, Gw_trend.T)

    def moe_out(m, Ew, Eb):
        return pl.pallas_call(
            functools.partial(_expert_kernel, m=m),
            grid=(E,),
            in_specs=[
                pl.BlockSpec((1, T, L), lambda e: (m, 0, 0)),
                pl.BlockSpec((1, P, L), lambda e: (e, 0, 0)),
                pl.BlockSpec((2, T, E), lambda e: (0, 0, 0)),
                pl.BlockSpec((E, P), lambda e: (0, 0)),
            ],
            out_specs=pl.BlockSpec((T, P), lambda e: (0, 0)),
            out_shape=jax.ShapeDtypeStruct((T, P), jnp.float32),
        )(st, Ew, c_all, Eb)

    out_tok = moe_out(0, Ew_sea, Eb_sea) + moe_out(1, Ew_trend, Eb_trend)
    out = jnp.transpose(out_tok.reshape(B, V, P), (0, 2, 1))
    return out, pm


# speedup vs baseline: 1.0691x; 1.0202x over previous
import functools

import jax
import jax.numpy as jnp
from jax.experimental import pallas as pl

KERNEL = 25
TOPK = 2


def _decomp_gate_kernel(x_ref, gs_ref, gt_ref, st_ref, c_ref, pm_ref, *, V, L, E, B):
    b = pl.program_id(0)
    xb = x_ref[...]
    pad = (KERNEL - 1) // 2
    front = jnp.broadcast_to(xb[:, :1], (V, pad))
    back = jnp.broadcast_to(xb[:, -1:], (V, pad))
    xp = jnp.concatenate([front, xb, back], axis=1)
    mov = xp[:, 0:L]
    for d in range(1, KERNEL):
        mov = mov + xp[:, d:d + L]
    mov = mov * (1.0 / KERNEL)
    sea = xb - mov
    st_ref[0] = sea.astype(jnp.bfloat16)
    st_ref[1] = mov.astype(jnp.bfloat16)

    for m, (a, g_ref) in enumerate(((sea, gs_ref), (mov, gt_ref))):
        logits = jnp.dot(a, g_ref[...], preferred_element_type=jnp.float32)
        z = logits - jnp.max(logits, axis=-1, keepdims=True)
        ez = jnp.exp(z)
        probs = ez / jnp.sum(ez, axis=-1, keepdims=True)
        iota = jax.lax.broadcasted_iota(jnp.int32, (V, E), 1)
        m1 = jnp.max(probs, axis=-1, keepdims=True)
        sel1 = jnp.min(jnp.where(probs == m1, iota, E), axis=-1, keepdims=True)
        oh1 = iota == sel1
        masked = jnp.where(oh1, -1.0, probs)
        m2 = jnp.max(masked, axis=-1, keepdims=True)
        sel2 = jnp.min(jnp.where(masked == m2, iota, E), axis=-1, keepdims=True)
        oh2 = iota == sel2
        c_ref[m] = jnp.where(oh1 | oh2, probs, 0.0)
        if m == 1:
            scaled = probs * (1.0 / B)

            @pl.when(b == 0)
            def _():
                pm_ref[...] = scaled

            @pl.when(b > 0)
            def _():
                pm_ref[...] += scaled


def _expert_kernel(a_ref, w_ref, c_ref, b_ref, o_ref, *, m):
    e = pl.program_id(0)
    y = jax.lax.dot_general(
        a_ref[0], w_ref[0].astype(jnp.bfloat16), (((1,), (1,)), ((), ())),
        preferred_element_type=jnp.float32)
    c_full = c_ref[m]
    lane = jax.lax.broadcasted_iota(jnp.int32, c_full.shape, 1)
    ccol = jnp.sum(jnp.where(lane == e, c_full, 0.0), axis=1, keepdims=True)
    b_full = b_ref[...]
    sub = jax.lax.broadcasted_iota(jnp.int32, b_full.shape, 0)
    brow = jnp.sum(jnp.where(sub == e, b_full, 0.0), axis=0, keepdims=True)
    contrib = ccol * (y + brow)

    @pl.when(e == 0)
    def _():
        o_ref[...] = contrib

    @pl.when(e > 0)
    def _():
        o_ref[...] += contrib


def kernel(x, Gw_sea, Ew_sea, Eb_sea, Gw_trend, Ew_trend, Eb_trend):
    B, L, V = x.shape
    E, P, _ = Ew_sea.shape
    T = B * V
    xt = jnp.transpose(x, (0, 2, 1)).reshape(T, L)

    st, c_all, pm = pl.pallas_call(
        functools.partial(_decomp_gate_kernel, V=V, L=L, E=E, B=B),
        grid=(B,),
        in_specs=[
            pl.BlockSpec((V, L), lambda b: (b, 0)),
            pl.BlockSpec((L, E), lambda b: (0, 0)),
            pl.BlockSpec((L, E), lambda b: (0, 0)),
        ],
        out_specs=[
            pl.BlockSpec((2, V, L), lambda b: (0, b, 0)),
            pl.BlockSpec((2, V, E), lambda b: (0, b, 0)),
            pl.BlockSpec((V, E), lambda b: (0, 0)),
        ],
        out_shape=[
            jax.ShapeDtypeStruct((2, T, L), jnp.bfloat16),
            jax.ShapeDtypeStruct((2, T, E), jnp.float32),
            jax.ShapeDtypeStruct((V, E), jnp.float32),
        ],
    )(xt, Gw_sea.T, Gw_trend.T)

    def moe_out(m, Ew, Eb):
        return pl.pallas_call(
            functools.partial(_expert_kernel, m=m),
            grid=(E,),
            in_specs=[
                pl.BlockSpec((1, T, L), lambda e: (m, 0, 0)),
                pl.BlockSpec((1, P, L), lambda e: (e, 0, 0)),
                pl.BlockSpec((2, T, E), lambda e: (0, 0, 0)),
                pl.BlockSpec((E, P), lambda e: (0, 0)),
            ],
            out_specs=pl.BlockSpec((T, P), lambda e: (0, 0)),
            out_shape=jax.ShapeDtypeStruct((T, P), jnp.float32),
        )(st, Ew, c_all, Eb)

    out_tok = moe_out(0, Ew_sea, Eb_sea) + moe_out(1, Ew_trend, Eb_trend)
    out = jnp.transpose(out_tok.reshape(B, V, P), (0, 2, 1))
    return out, pm


# speedup vs baseline: 1.9382x; 1.8130x over previous
import functools

import jax
import jax.numpy as jnp
from jax.experimental import pallas as pl

KERNEL = 25
TOPK = 2


def _decomp_gate_kernel(x_ref, gs_ref, gt_ref, st_ref, c_ref, pm_ref, *, V, L, E, B):
    b = pl.program_id(0)
    xb = x_ref[0]
    pad = (KERNEL - 1) // 2
    front = jnp.broadcast_to(xb[:1, :], (pad, V))
    back = jnp.broadcast_to(xb[-1:, :], (pad, V))
    xp = jnp.concatenate([front, xb, back], axis=0)
    a3 = xp[0:L + 7] + xp[8:L + 15] + xp[16:L + 23]
    mov = a3[0:L]
    for r in range(1, 8):
        mov = mov + a3[r:r + L]
    mov = (mov + xp[24:L + 24]) * (1.0 / KERNEL)
    sea = xb - mov
    st_ref[0] = sea.astype(jnp.bfloat16)
    st_ref[1] = mov.astype(jnp.bfloat16)

    for m, (a, g_ref) in enumerate(((sea, gs_ref), (mov, gt_ref))):
        logits = jax.lax.dot_general(
            g_ref[...], a, (((1,), (0,)), ((), ())),
            preferred_element_type=jnp.float32)
        z = logits - jnp.max(logits, axis=0, keepdims=True)
        ez = jnp.exp(z)
        probs = ez / jnp.sum(ez, axis=0, keepdims=True)
        iota = jax.lax.broadcasted_iota(jnp.int32, (E, V), 0)
        m1 = jnp.max(probs, axis=0, keepdims=True)
        sel1 = jnp.min(jnp.where(probs == m1, iota, E), axis=0, keepdims=True)
        oh1 = iota == sel1
        masked = jnp.where(oh1, -1.0, probs)
        m2 = jnp.max(masked, axis=0, keepdims=True)
        sel2 = jnp.min(jnp.where(masked == m2, iota, E), axis=0, keepdims=True)
        oh2 = iota == sel2
        c_ref[m] = jnp.where(oh1 | oh2, probs, 0.0)
        if m == 1:
            scaled = probs * (1.0 / B)

            @pl.when(b == 0)
            def _():
                pm_ref[...] = scaled

            @pl.when(b > 0)
            def _():
                pm_ref[...] += scaled


def _expert_kernel(a_ref, w_ref, c_ref, bt_ref, o_ref, *, m, E):
    e = pl.program_id(0)
    y = jax.lax.dot_general(
        w_ref[0].astype(jnp.bfloat16), a_ref[0], (((1,), (0,)), ((), ())),
        preferred_element_type=jnp.float32)
    cm = c_ref[m]
    sub = jax.lax.broadcasted_iota(jnp.int32, cm.shape, 0)
    crow = jnp.sum(jnp.where(sub == e, cm, 0.0), axis=0, keepdims=True)
    bt = bt_ref[...]
    lane = jax.lax.broadcasted_iota(jnp.int32, bt.shape, 1)
    bcol = jnp.sum(jnp.where(lane == e, bt, 0.0), axis=1, keepdims=True)
    contrib = crow * (y + bcol)

    @pl.when(e == 0)
    def _():
        o_ref[...] = contrib

    @pl.when(e > 0)
    def _():
        o_ref[...] += contrib


def kernel(x, Gw_sea, Ew_sea, Eb_sea, Gw_trend, Ew_trend, Eb_trend):
    B, L, V = x.shape
    E, P, _ = Ew_sea.shape
    T = B * V

    st, c_all, pm = pl.pallas_call(
        functools.partial(_decomp_gate_kernel, V=V, L=L, E=E, B=B),
        grid=(B,),
        in_specs=[
            pl.BlockSpec((1, L, V), lambda b: (b, 0, 0)),
            pl.BlockSpec((E, L), lambda b: (0, 0)),
            pl.BlockSpec((E, L), lambda b: (0, 0)),
        ],
        out_specs=[
            pl.BlockSpec((2, L, V), lambda b: (0, 0, b)),
            pl.BlockSpec((2, E, V), lambda b: (0, 0, b)),
            pl.BlockSpec((E, V), lambda b: (0, 0)),
        ],
        out_shape=[
            jax.ShapeDtypeStruct((2, L, T), jnp.bfloat16),
            jax.ShapeDtypeStruct((2, E, T), jnp.float32),
            jax.ShapeDtypeStruct((E, V), jnp.float32),
        ],
    )(x, Gw_sea, Gw_trend)

    def moe_out(m, Ew, Eb):
        return pl.pallas_call(
            functools.partial(_expert_kernel, m=m, E=E),
            grid=(E,),
            in_specs=[
                pl.BlockSpec((1, L, T), lambda e: (m, 0, 0)),
                pl.BlockSpec((1, P, L), lambda e: (e, 0, 0)),
                pl.BlockSpec((2, E, T), lambda e: (0, 0, 0)),
                pl.BlockSpec((P, E), lambda e: (0, 0)),
            ],
            out_specs=pl.BlockSpec((P, T), lambda e: (0, 0)),
            out_shape=jax.ShapeDtypeStruct((P, T), jnp.float32),
        )(st, Ew, c_all, Eb.T)

    out_t = moe_out(0, Ew_sea, Eb_sea) + moe_out(1, Ew_trend, Eb_trend)
    out = jnp.transpose(out_t.reshape(P, B, V), (1, 0, 2))
    return out, pm.T


# speedup vs baseline: 2.0864x; 1.0764x over previous
import functools

import jax
import jax.numpy as jnp
from jax.experimental import pallas as pl
from jax.experimental.pallas import tpu as pltpu

KERNEL = 25
TOPK = 2


def _fused_kernel(x_ref, gs_ref, gt_ref, ws_ref, wt_ref, bts_ref, btt_ref,
                  o_ref, pm_ref, a_scr, c_scr, *, B, V, L, E, P):
    m = pl.program_id(0)
    e = pl.program_id(1)
    ph = pl.program_id(2)
    P2 = P // 2
    first = (m == 0) & (e == 0)

    @pl.when(first & (ph == 0))
    def _prologue():
        pad = (KERNEL - 1) // 2
        pm_sum = jnp.zeros((E, V), jnp.float32)
        for b in range(B):
            xb = x_ref[b]
            front = jnp.broadcast_to(xb[:1, :], (pad, V))
            back = jnp.broadcast_to(xb[-1:, :], (pad, V))
            xp = jnp.concatenate([front, xb, back], axis=0)
            a3 = xp[0:L + 7] + xp[8:L + 15] + xp[16:L + 23]
            mov = a3[0:L]
            for r in range(1, 8):
                mov = mov + a3[r:r + L]
            mov = (mov + xp[24:L + 24]) * (1.0 / KERNEL)
            sea = xb - mov
            sl = slice(b * V, (b + 1) * V)
            a_scr[0, :, sl] = sea.astype(jnp.bfloat16)
            a_scr[1, :, sl] = mov.astype(jnp.bfloat16)
            for mi, (a, g_ref) in enumerate(((sea, gs_ref), (mov, gt_ref))):
                logits = jax.lax.dot_general(
                    g_ref[...], a, (((1,), (0,)), ((), ())),
                    preferred_element_type=jnp.float32)
                z = logits - jnp.max(logits, axis=0, keepdims=True)
                ez = jnp.exp(z)
                probs = ez / jnp.sum(ez, axis=0, keepdims=True)
                iota = jax.lax.broadcasted_iota(jnp.int32, (E, V), 0)
                m1 = jnp.max(probs, axis=0, keepdims=True)
                sel1 = jnp.min(jnp.where(probs == m1, iota, E), axis=0,
                               keepdims=True)
                oh1 = iota == sel1
                masked = jnp.where(oh1, -1.0, probs)
                m2 = jnp.max(masked, axis=0, keepdims=True)
                sel2 = jnp.min(jnp.where(masked == m2, iota, E), axis=0,
                               keepdims=True)
                oh2 = iota == sel2
                c_scr[mi, :, sl] = jnp.where(oh1 | oh2, probs, 0.0)
                if mi == 1:
                    pm_sum = pm_sum + probs * (1.0 / B)
        pm_ref[...] = pm_sum

    def accumulate(mi, w_ref, bt_ref):
        y = jax.lax.dot_general(
            w_ref[0].astype(jnp.bfloat16), a_scr[mi],
            (((1,), (0,)), ((), ())),
            preferred_element_type=jnp.float32)
        cm = c_scr[mi]
        sub = jax.lax.broadcasted_iota(jnp.int32, cm.shape, 0)
        crow = jnp.sum(jnp.where(sub == e, cm, 0.0), axis=0, keepdims=True)
        prow = pl.multiple_of(ph * P2, P2)
        bt = bt_ref[pl.ds(prow, P2), :]
        lane = jax.lax.broadcasted_iota(jnp.int32, bt.shape, 1)
        bcol = jnp.sum(jnp.where(lane == e, bt, 0.0), axis=1, keepdims=True)
        contrib = crow * (y + bcol)
        for b in range(B):
            blk = contrib[:, b * V:(b + 1) * V]

            @pl.when(first)
            def _():
                o_ref[b, pl.ds(prow, P2), :] = blk

            @pl.when(~first)
            def _():
                o_ref[b, pl.ds(prow, P2), :] += blk

    @pl.when(m == 0)
    def _():
        accumulate(0, ws_ref, bts_ref)

    @pl.when(m == 1)
    def _():
        accumulate(1, wt_ref, btt_ref)


def kernel(x, Gw_sea, Ew_sea, Eb_sea, Gw_trend, Ew_trend, Eb_trend):
    B, L, V = x.shape
    E, P, _ = Ew_sea.shape
    T = B * V

    out, pm = pl.pallas_call(
        functools.partial(_fused_kernel, B=B, V=V, L=L, E=E, P=P),
        grid=(2, E, 2),
        in_specs=[
            pl.BlockSpec((B, L, V), lambda m, e, ph: (0, 0, 0)),
            pl.BlockSpec((E, L), lambda m, e, ph: (0, 0)),
            pl.BlockSpec((E, L), lambda m, e, ph: (0, 0)),
            pl.BlockSpec((1, P // 2, L),
                         lambda m, e, ph: ((1 - m) * e + m * (E - 1),
                                           (1 - m) * ph + m, 0)),
            pl.BlockSpec((1, P // 2, L),
                         lambda m, e, ph: (m * e, m * ph, 0)),
            pl.BlockSpec((P, E), lambda m, e, ph: (0, 0)),
            pl.BlockSpec((P, E), lambda m, e, ph: (0, 0)),
        ],
        out_specs=[
            pl.BlockSpec((B, P, V), lambda m, e, ph: (0, 0, 0)),
            pl.BlockSpec((E, V), lambda m, e, ph: (0, 0)),
        ],
        out_shape=[
            jax.ShapeDtypeStruct((B, P, V), jnp.float32),
            jax.ShapeDtypeStruct((E, V), jnp.float32),
        ],
        scratch_shapes=[
            pltpu.VMEM((2, L, T), jnp.bfloat16),
            pltpu.VMEM((2, E, T), jnp.float32),
        ],
    )(x, Gw_sea, Gw_trend, Ew_sea, Ew_trend, Eb_sea.T, Eb_trend.T)

    return out, pm.T
